# baseline (speedup 1.0000x reference)
import functools

import jax
import jax.numpy as jnp
from jax import lax
from jax.experimental import pallas as pl
from jax.experimental.pallas import tpu as pltpu
from jax.experimental.pallas import tpu_sc as plsc

DIM = 64
N_EMBED = 1024


_BLK = 2048


def _dist_argmin_kernel(x_ref, e_ref, ind_ref, diff_ref, em2_ref, e2_ref, io_ref):
    i = pl.program_id(0)
    nsteps = pl.num_programs(0)

    @pl.when(i == 0)
    def _():
        e = e_ref[...]
        em2_ref[...] = e * (-2.0)
        e2_ref[...] = jnp.sum(e * e, axis=0, keepdims=True)
        io_ref[...] = lax.broadcasted_iota(jnp.int32, (1, N_EMBED), 1).astype(
            jnp.float32
        )

    x = x_ref[...]
    xe2 = jnp.dot(x, em2_ref[...], preferred_element_type=jnp.float32)
    s = xe2 + e2_ref[...]
    minv0 = jnp.min(s, axis=1)
    idxf = jnp.min(
        jnp.where(s == minv0[:, None], io_ref[...], float(N_EMBED)), axis=1
    )
    idx = idxf.astype(jnp.int32)
    x2v = jnp.sum(x * x, axis=1)
    minv = minv0 + x2v
    ind_ref[0, 0, :] = idx
    part = jnp.sum(jnp.maximum(minv, 0.0), keepdims=True).reshape(1, 1)

    @pl.when(i == 0)
    def _():
        diff_ref[...] = jnp.zeros((1, 1), jnp.float32)

    diff_ref[...] += part


_TOTAL_ELEMS = 64 * 1024 * DIM


def _dist_argmin(x, embed):
    n = x.shape[0]
    nblk = n // _BLK
    return pl.pallas_call(
        _dist_argmin_kernel,
        grid=(nblk,),
        in_specs=[
            pl.BlockSpec((_BLK, DIM), lambda i: (i, 0)),
            pl.BlockSpec((DIM, N_EMBED), lambda i: (0, 0)),
        ],
        out_specs=[
            pl.BlockSpec((1, 1, _BLK), lambda i: (i, 0, 0)),
            pl.BlockSpec((1, 1), lambda i: (0, 0)),
        ],
        out_shape=[
            jax.ShapeDtypeStruct((nblk, 1, _BLK), jnp.int32),
            jax.ShapeDtypeStruct((1, 1), jnp.float32),
        ],
        scratch_shapes=[
            pltpu.VMEM((DIM, N_EMBED), jnp.float32),
            pltpu.VMEM((1, N_EMBED), jnp.float32),
            pltpu.VMEM((1, N_EMBED), jnp.float32),
        ],
    )(x, embed)


_SC_CHUNK = 256


def _make_sc_gather(n):
    info = plsc.get_sparse_core_info()
    nc, ns = info.num_cores, info.num_subcores
    nw = nc * ns
    b_per_w = n // nw
    nchunk = b_per_w // _SC_CHUNK
    mesh = plsc.VectorSubcoreMesh(core_axis_name="c", subcore_axis_name="s")

    @functools.partial(
        pl.kernel,
        mesh=mesh,
        compiler_params=pltpu.CompilerParams(use_tc_tiling_on_sc=False),
        out_type=jax.ShapeDtypeStruct((n * DIM,), jnp.float32),
        scratch_types=[
            pltpu.VMEM((N_EMBED * DIM,), jnp.float32),
            pltpu.VMEM((b_per_w,), jnp.int32),
            pltpu.VMEM((2, _SC_CHUNK * DIM), jnp.float32),
            pltpu.SemaphoreType.DMA,
            pltpu.SemaphoreType.DMA,
            [pltpu.SemaphoreType.DMA] * 2,
        ],
    )
    def sc_gather(table_hbm, idx_hbm, out_hbm, table_v, idx_v, out_v, st, si, sw):
        wid = lax.axis_index("s") * nc + lax.axis_index("c")
        base = wid * b_per_w
        tcp = pltpu.async_copy(table_hbm, table_v, st)
        icp = pltpu.async_copy(idx_hbm.at[pl.ds(base, b_per_w)], idx_v, si)
        tcp.wait()
        icp.wait()
        wcp = [None] * nchunk
        for c in range(nchunk):
            b = c % 2
            if c >= 2:
                wcp[c - 2].wait()

            @plsc.parallel_loop(0, _SC_CHUNK // 16, 1, unroll=2)
            def _(g):
                av = idx_v[pl.ds(c * _SC_CHUNK + g * 16, 16)] * DIM
                tbase = g * (16 * DIM)
                for k in range(16):
                    a = av[k]
                    for j in range(0, DIM, 16):
                        out_v[b, pl.ds(tbase + k * DIM + j, 16)] = (
                            table_v[pl.ds(a + j, 16)]
                        )

            wcp[c] = pltpu.async_copy(
                out_v.at[b],
                out_hbm.at[pl.ds((base + c * _SC_CHUNK) * DIM, _SC_CHUNK * DIM)],
                sw[b],
            )
        wcp[nchunk - 2].wait()
        wcp[nchunk - 1].wait()

    return sc_gather


_NSLICE = 2


def kernel(input, embed):
    dim = embed.shape[0]
    x = input.reshape(-1, dim)
    n = x.shape[0]
    ns = n // _NSLICE
    table = embed.T.reshape(-1)
    sc_gather = _make_sc_gather(ns)
    inds, qs, dsums = [], [], []
    for si in range(_NSLICE):
        ind3, dsum11 = _dist_argmin(lax.slice(x, (si * ns, 0), ((si + 1) * ns, dim)), embed)
        ind_flat = ind3.reshape(-1)
        qs.append(sc_gather(table, ind_flat))
        inds.append(ind_flat)
        dsums.append(dsum11)
    q = jnp.concatenate(qs)
    quantize = q.reshape(input.shape)
    diff = (sum(dsums) / float(n * dim)).reshape(())
    embed_ind = jnp.concatenate(inds).reshape(input.shape[:-1])
    return (quantize, diff, embed_ind)


# speedup vs baseline: 1.4642x; 1.4642x over previous
import functools

import jax
import jax.numpy as jnp
from jax import lax
from jax.experimental import pallas as pl
from jax.experimental.pallas import tpu as pltpu
from jax.experimental.pallas import tpu_sc as plsc

DIM = 64
N_EMBED = 1024


_BLK = 2048


def _dist_argmin_kernel(x_ref, e_ref, ind_ref, diff_ref, em2_ref, e2_ref, io_ref):
    i = pl.program_id(0)
    nsteps = pl.num_programs(0)

    @pl.when(i == 0)
    def _():
        e = e_ref[...]
        em2_ref[...] = e * (-2.0)
        e2_ref[...] = jnp.sum(e * e, axis=0, keepdims=True)
        io_ref[...] = lax.broadcasted_iota(jnp.int32, (1, N_EMBED), 1).astype(
            jnp.float32
        )

    x = x_ref[...]
    xe2 = jnp.dot(x, em2_ref[...], preferred_element_type=jnp.float32)
    s = xe2 + e2_ref[...]
    minv0 = jnp.min(s, axis=1)
    idxf = jnp.min(
        jnp.where(s == minv0[:, None], io_ref[...], float(N_EMBED)), axis=1
    )
    idx = idxf.astype(jnp.int32)
    x2v = jnp.sum(x * x, axis=1)
    minv = minv0 + x2v
    ind_ref[...] = idx.reshape(_BLK // 128, 128)
    part = jnp.sum(jnp.maximum(minv, 0.0), keepdims=True).reshape(1, 1)

    @pl.when(i == 0)
    def _():
        diff_ref[...] = jnp.zeros((1, 1), jnp.float32)

    diff_ref[...] += part


_TOTAL_ELEMS = 64 * 1024 * DIM


def _dist_argmin(x, embed):
    n = x.shape[0]
    nblk = n // _BLK
    return pl.pallas_call(
        _dist_argmin_kernel,
        grid=(nblk,),
        in_specs=[
            pl.BlockSpec((_BLK, DIM), lambda i: (i, 0)),
            pl.BlockSpec((DIM, N_EMBED), lambda i: (0, 0)),
        ],
        out_specs=[
            pl.BlockSpec((_BLK // 128, 128), lambda i: (i, 0)),
            pl.BlockSpec((1, 1), lambda i: (0, 0)),
        ],
        out_shape=[
            jax.ShapeDtypeStruct((n // 128, 128), jnp.int32),
            jax.ShapeDtypeStruct((1, 1), jnp.float32),
        ],
        scratch_shapes=[
            pltpu.VMEM((DIM, N_EMBED), jnp.float32),
            pltpu.VMEM((1, N_EMBED), jnp.float32),
            pltpu.VMEM((1, N_EMBED), jnp.float32),
        ],
    )(x, embed)


_SC_CHUNK = 256


def _make_sc_gather(n):
    info = plsc.get_sparse_core_info()
    nc, ns = info.num_cores, info.num_subcores
    nw = nc * ns
    b_per_w = n // nw
    nchunk = b_per_w // _SC_CHUNK
    mesh = plsc.VectorSubcoreMesh(core_axis_name="c", subcore_axis_name="s")

    @functools.partial(
        pl.kernel,
        mesh=mesh,
        compiler_params=pltpu.CompilerParams(use_tc_tiling_on_sc=False),
        out_type=jax.ShapeDtypeStruct((n * DIM,), jnp.float32),
        scratch_types=[
            pltpu.VMEM((N_EMBED * DIM,), jnp.float32),
            pltpu.VMEM((b_per_w,), jnp.int32),
            pltpu.VMEM((2, _SC_CHUNK * DIM), jnp.float32),
            pltpu.SemaphoreType.DMA,
            pltpu.SemaphoreType.DMA,
            [pltpu.SemaphoreType.DMA] * 2,
        ],
    )
    def sc_gather(table_hbm, idx_hbm, out_hbm, table_v, idx_v, out_v, st, si, sw):
        wid = lax.axis_index("s") * nc + lax.axis_index("c")
        base = wid * b_per_w
        tcp = pltpu.async_copy(table_hbm, table_v, st)
        icp = pltpu.async_copy(idx_hbm.at[pl.ds(base, b_per_w)], idx_v, si)
        tcp.wait()
        icp.wait()
        wcp = [None] * nchunk
        for c in range(nchunk):
            b = c % 2
            if c >= 2:
                wcp[c - 2].wait()

            @plsc.parallel_loop(0, _SC_CHUNK // 16, 1, unroll=2)
            def _(g):
                av = idx_v[pl.ds(c * _SC_CHUNK + g * 16, 16)] * DIM
                tbase = g * (16 * DIM)
                for k in range(16):
                    a = av[k]
                    for j in range(0, DIM, 16):
                        out_v[b, pl.ds(tbase + k * DIM + j, 16)] = (
                            table_v[pl.ds(a + j, 16)]
                        )

            wcp[c] = pltpu.async_copy(
                out_v.at[b],
                out_hbm.at[pl.ds((base + c * _SC_CHUNK) * DIM, _SC_CHUNK * DIM)],
                sw[b],
            )
        wcp[nchunk - 2].wait()
        wcp[nchunk - 1].wait()

    return sc_gather


_NSLICE = 1


def kernel(input, embed):
    dim = embed.shape[0]
    x = input.reshape(-1, dim)
    n = x.shape[0]
    ns = n // _NSLICE
    table = embed.T.reshape(-1)
    sc_gather = _make_sc_gather(ns)
    inds, qs, dsums = [], [], []
    for si in range(_NSLICE):
        ind3, dsum11 = _dist_argmin(lax.slice(x, (si * ns, 0), ((si + 1) * ns, dim)), embed)
        ind_flat = ind3.reshape(-1)
        qs.append(sc_gather(table, ind_flat))
        inds.append(ind_flat)
        dsums.append(dsum11)
    q = jnp.concatenate(qs)
    quantize = q.reshape(input.shape)
    diff = (sum(dsums) / float(n * dim)).reshape(())
    embed_ind = jnp.concatenate(inds).reshape(input.shape[:-1])
    return (quantize, diff, embed_ind)


# speedup vs baseline: 1.4980x; 1.0231x over previous
import functools

import jax
import jax.numpy as jnp
from jax import lax
from jax.experimental import pallas as pl
from jax.experimental.pallas import tpu as pltpu
from jax.experimental.pallas import tpu_sc as plsc

DIM = 64
N_EMBED = 1024


_BLK = 4096


def _dist_argmin_kernel(x_ref, e_ref, ind_ref, diff_ref, em2_ref, e2_ref, io_ref):
    i = pl.program_id(0)
    nsteps = pl.num_programs(0)

    @pl.when(i == 0)
    def _():
        e = e_ref[...]
        em2_ref[...] = e * (-2.0)
        e2_ref[...] = jnp.sum(e * e, axis=0, keepdims=True)
        io_ref[...] = lax.broadcasted_iota(jnp.int32, (1, N_EMBED), 1).astype(
            jnp.float32
        )

    x = x_ref[...]
    xe2 = jnp.dot(x, em2_ref[...], preferred_element_type=jnp.float32)
    s = xe2 + e2_ref[...]
    minv0 = jnp.min(s, axis=1)
    idxf = jnp.min(
        jnp.where(s == minv0[:, None], io_ref[...], float(N_EMBED)), axis=1
    )
    idx = idxf.astype(jnp.int32)
    x2v = jnp.sum(x * x, axis=1)
    minv = minv0 + x2v
    ind_ref[...] = idx.reshape(_BLK // 128, 128)
    part = jnp.sum(jnp.maximum(minv, 0.0), keepdims=True).reshape(1, 1)

    @pl.when(i == 0)
    def _():
        diff_ref[...] = jnp.zeros((1, 1), jnp.float32)

    diff_ref[...] += part


_TOTAL_ELEMS = 64 * 1024 * DIM


def _dist_argmin(x, embed):
    n = x.shape[0]
    nblk = n // _BLK
    return pl.pallas_call(
        _dist_argmin_kernel,
        grid=(nblk,),
        in_specs=[
            pl.BlockSpec((_BLK, DIM), lambda i: (i, 0)),
            pl.BlockSpec((DIM, N_EMBED), lambda i: (0, 0)),
        ],
        out_specs=[
            pl.BlockSpec((_BLK // 128, 128), lambda i: (i, 0)),
            pl.BlockSpec((1, 1), lambda i: (0, 0)),
        ],
        out_shape=[
            jax.ShapeDtypeStruct((n // 128, 128), jnp.int32),
            jax.ShapeDtypeStruct((1, 1), jnp.float32),
        ],
        scratch_shapes=[
            pltpu.VMEM((DIM, N_EMBED), jnp.float32),
            pltpu.VMEM((1, N_EMBED), jnp.float32),
            pltpu.VMEM((1, N_EMBED), jnp.float32),
        ],
    )(x, embed)


_SC_CHUNK = 256


def _make_sc_gather(n):
    info = plsc.get_sparse_core_info()
    nc, ns = info.num_cores, info.num_subcores
    nw = nc * ns
    b_per_w = n // nw
    nchunk = b_per_w // _SC_CHUNK
    mesh = plsc.VectorSubcoreMesh(core_axis_name="c", subcore_axis_name="s")

    @functools.partial(
        pl.kernel,
        mesh=mesh,
        compiler_params=pltpu.CompilerParams(use_tc_tiling_on_sc=False),
        out_type=jax.ShapeDtypeStruct((n * DIM,), jnp.float32),
        scratch_types=[
            pltpu.VMEM((N_EMBED * DIM,), jnp.float32),
            pltpu.VMEM((b_per_w,), jnp.int32),
            pltpu.VMEM((2, _SC_CHUNK * DIM), jnp.float32),
            pltpu.SemaphoreType.DMA,
            pltpu.SemaphoreType.DMA,
            [pltpu.SemaphoreType.DMA] * 2,
        ],
    )
    def sc_gather(table_hbm, idx_hbm, out_hbm, table_v, idx_v, out_v, st, si, sw):
        wid = lax.axis_index("s") * nc + lax.axis_index("c")
        base = wid * b_per_w
        tcp = pltpu.async_copy(table_hbm, table_v, st)
        icp = pltpu.async_copy(idx_hbm.at[pl.ds(base, b_per_w)], idx_v, si)
        tcp.wait()
        icp.wait()
        wcp = [None] * nchunk
        for c in range(nchunk):
            b = c % 2
            if c >= 2:
                wcp[c - 2].wait()

            @plsc.parallel_loop(0, _SC_CHUNK // 16, 1, unroll=2)
            def _(g):
                av = idx_v[pl.ds(c * _SC_CHUNK + g * 16, 16)] * DIM
                tbase = g * (16 * DIM)
                for k in range(16):
                    a = av[k]
                    for j in range(0, DIM, 16):
                        out_v[b, pl.ds(tbase + k * DIM + j, 16)] = (
                            table_v[pl.ds(a + j, 16)]
                        )

            wcp[c] = pltpu.async_copy(
                out_v.at[b],
                out_hbm.at[pl.ds((base + c * _SC_CHUNK) * DIM, _SC_CHUNK * DIM)],
                sw[b],
            )
        wcp[nchunk - 2].wait()
        wcp[nchunk - 1].wait()

    return sc_gather


_NSLICE = 1


def kernel(input, embed):
    dim = embed.shape[0]
    x = input.reshape(-1, dim)
    n = x.shape[0]
    ns = n // _NSLICE
    table = embed.T.reshape(-1)
    sc_gather = _make_sc_gather(ns)
    inds, qs, dsums = [], [], []
    for si in range(_NSLICE):
        ind3, dsum11 = _dist_argmin(lax.slice(x, (si * ns, 0), ((si + 1) * ns, dim)), embed)
        ind_flat = ind3.reshape(-1)
        qs.append(sc_gather(table, ind_flat))
        inds.append(ind_flat)
        dsums.append(dsum11)
    q = jnp.concatenate(qs)
    quantize = q.reshape(input.shape)
    diff = (sum(dsums) / float(n * dim)).reshape(())
    embed_ind = jnp.concatenate(inds).reshape(input.shape[:-1])
    return (quantize, diff, embed_ind)


# speedup vs baseline: 1.5243x; 1.0176x over previous
import functools

import jax
import jax.numpy as jnp
from jax import lax
from jax.experimental import pallas as pl
from jax.experimental.pallas import tpu as pltpu
from jax.experimental.pallas import tpu_sc as plsc

DIM = 64
N_EMBED = 1024


_BLK = 8192


def _dist_argmin_kernel(x_ref, e_ref, ind_ref, diff_ref, em2_ref, e2_ref, io_ref):
    i = pl.program_id(0)
    nsteps = pl.num_programs(0)

    @pl.when(i == 0)
    def _():
        e = e_ref[...]
        em2_ref[...] = e * (-2.0)
        e2_ref[...] = jnp.sum(e * e, axis=0, keepdims=True)
        io_ref[...] = lax.broadcasted_iota(jnp.int32, (1, N_EMBED), 1).astype(
            jnp.float32
        )

    x = x_ref[...]
    xe2 = jnp.dot(x, em2_ref[...], preferred_element_type=jnp.float32)
    s = xe2 + e2_ref[...]
    minv0 = jnp.min(s, axis=1)
    idxf = jnp.min(
        jnp.where(s == minv0[:, None], io_ref[...], float(N_EMBED)), axis=1
    )
    idx = idxf.astype(jnp.int32)
    x2v = jnp.sum(x * x, axis=1)
    minv = minv0 + x2v
    ind_ref[...] = idx.reshape(_BLK // 128, 128)
    part = jnp.sum(jnp.maximum(minv, 0.0), keepdims=True).reshape(1, 1)

    @pl.when(i == 0)
    def _():
        diff_ref[...] = jnp.zeros((1, 1), jnp.float32)

    diff_ref[...] += part


_TOTAL_ELEMS = 64 * 1024 * DIM


def _dist_argmin(x, embed):
    n = x.shape[0]
    nblk = n // _BLK
    return pl.pallas_call(
        _dist_argmin_kernel,
        grid=(nblk,),
        in_specs=[
            pl.BlockSpec((_BLK, DIM), lambda i: (i, 0)),
            pl.BlockSpec((DIM, N_EMBED), lambda i: (0, 0)),
        ],
        out_specs=[
            pl.BlockSpec((_BLK // 128, 128), lambda i: (i, 0)),
            pl.BlockSpec((1, 1), lambda i: (0, 0)),
        ],
        out_shape=[
            jax.ShapeDtypeStruct((n // 128, 128), jnp.int32),
            jax.ShapeDtypeStruct((1, 1), jnp.float32),
        ],
        scratch_shapes=[
            pltpu.VMEM((DIM, N_EMBED), jnp.float32),
            pltpu.VMEM((1, N_EMBED), jnp.float32),
            pltpu.VMEM((1, N_EMBED), jnp.float32),
        ],
    )(x, embed)


_SC_CHUNK = 256


def _make_sc_gather(n):
    info = plsc.get_sparse_core_info()
    nc, ns = info.num_cores, info.num_subcores
    nw = nc * ns
    b_per_w = n // nw
    nchunk = b_per_w // _SC_CHUNK
    mesh = plsc.VectorSubcoreMesh(core_axis_name="c", subcore_axis_name="s")

    @functools.partial(
        pl.kernel,
        mesh=mesh,
        compiler_params=pltpu.CompilerParams(use_tc_tiling_on_sc=False),
        out_type=jax.ShapeDtypeStruct((n * DIM,), jnp.float32),
        scratch_types=[
            pltpu.VMEM((N_EMBED * DIM,), jnp.float32),
            pltpu.VMEM((b_per_w,), jnp.int32),
            pltpu.VMEM((2, _SC_CHUNK * DIM), jnp.float32),
            pltpu.SemaphoreType.DMA,
            pltpu.SemaphoreType.DMA,
            [pltpu.SemaphoreType.DMA] * 2,
        ],
    )
    def sc_gather(table_hbm, idx_hbm, out_hbm, table_v, idx_v, out_v, st, si, sw):
        wid = lax.axis_index("s") * nc + lax.axis_index("c")
        base = wid * b_per_w
        tcp = pltpu.async_copy(table_hbm, table_v, st)
        icp = pltpu.async_copy(idx_hbm.at[pl.ds(base, b_per_w)], idx_v, si)
        tcp.wait()
        icp.wait()
        wcp = [None] * nchunk
        for c in range(nchunk):
            b = c % 2
            if c >= 2:
                wcp[c - 2].wait()

            @plsc.parallel_loop(0, _SC_CHUNK // 16, 1, unroll=2)
            def _(g):
                av = idx_v[pl.ds(c * _SC_CHUNK + g * 16, 16)] * DIM
                tbase = g * (16 * DIM)
                for k in range(16):
                    a = av[k]
                    for j in range(0, DIM, 16):
                        out_v[b, pl.ds(tbase + k * DIM + j, 16)] = (
                            table_v[pl.ds(a + j, 16)]
                        )

            wcp[c] = pltpu.async_copy(
                out_v.at[b],
                out_hbm.at[pl.ds((base + c * _SC_CHUNK) * DIM, _SC_CHUNK * DIM)],
                sw[b],
            )
        wcp[nchunk - 2].wait()
        wcp[nchunk - 1].wait()

    return sc_gather


_NSLICE = 1


def kernel(input, embed):
    dim = embed.shape[0]
    x = input.reshape(-1, dim)
    n = x.shape[0]
    ns = n // _NSLICE
    table = embed.T.reshape(-1)
    sc_gather = _make_sc_gather(ns)
    inds, qs, dsums = [], [], []
    for si in range(_NSLICE):
        ind3, dsum11 = _dist_argmin(lax.slice(x, (si * ns, 0), ((si + 1) * ns, dim)), embed)
        ind_flat = ind3.reshape(-1)
        qs.append(sc_gather(table, ind_flat))
        inds.append(ind_flat)
        dsums.append(dsum11)
    q = jnp.concatenate(qs)
    quantize = q.reshape(input.shape)
    diff = (sum(dsums) / float(n * dim)).reshape(())
    embed_ind = jnp.concatenate(inds).reshape(input.shape[:-1])
    return (quantize, diff, embed_ind)
